# baseline (device time: 23035 ns/iter reference)
import functools

import jax
import jax.numpy as jnp
from jax import lax
from jax.experimental import pallas as pl
from jax.experimental.pallas import tpu as pltpu

N_DEV = 32
STAGE_OFFSETS = ((1, 2, 3), (4, 8, 12), (16,))
NEAR_OFFSETS = STAGE_OFFSETS[0]
FAR_OFFSETS = tuple(o for offs in STAGE_OFFSETS[1:] for o in offs)
N_STAGES = len(STAGE_OFFSETS)
MAX_RADIX = 3
N_CHUNKS = 2
EPS = 1e-5


def kernel(x, t_emb, W_scale, W_shift):
    b, s, c = x.shape
    c_global = c * N_DEV
    s2 = s // N_CHUNKS

    def body(x_ref, t_ref, ws_ref, wsh_ref, out_ref,
             xv_ref, ov_ref, acc_ref, recv_ref,
             load_sems, store_sems, send_sems, recv_sems):
        my = lax.axis_index("i")
        barrier_sem = pltpu.get_barrier_semaphore()

        @functools.partial(pl.run_scoped, far_sem=pltpu.SemaphoreType.REGULAR)
        def _(far_sem):
            for off in NEAR_OFFSETS:
                pl.semaphore_signal(
                    barrier_sem, inc=1,
                    device_id=(my ^ off,),
                    device_id_type=pl.DeviceIdType.MESH,
                )
            for off in FAR_OFFSETS:
                pl.semaphore_signal(
                    far_sem, inc=1,
                    device_id=(my ^ off,),
                    device_id_type=pl.DeviceIdType.MESH,
                )

            def chunk_slice(ref, q):
                return ref.at[:, q * s2:(q + 1) * s2, :]

            loads = []
            for q in range(N_CHUNKS):
                cp = pltpu.make_async_copy(
                    chunk_slice(x_ref, q), xv_ref.at[q], load_sems.at[q])
                cp.start()
                loads.append(cp)

            def stats(q):
                loads[q].wait()
                xq = xv_ref[q]
                acc_ref[q, 0] = jnp.sum(xq, axis=-1).astype(jnp.bfloat16)
                acc_ref[q, 1] = jnp.sum(xq * xq, axis=-1).astype(jnp.bfloat16)

            def start_stage(q, st):
                rdmas = []
                for p, off in enumerate(STAGE_OFFSETS[st]):
                    rdma = pltpu.make_async_remote_copy(
                        src_ref=acc_ref.at[q],
                        dst_ref=recv_ref.at[q, st, p],
                        send_sem=send_sems.at[q, st, p],
                        recv_sem=recv_sems.at[q, st, p],
                        device_id=(my ^ off,),
                        device_id_type=pl.DeviceIdType.MESH,
                    )
                    rdma.start()
                    rdmas.append(rdma)
                return rdmas

            def finish_stage(q, st, rdmas):
                for rdma in rdmas:
                    rdma.wait()
                upd = acc_ref[q]
                for p in range(len(STAGE_OFFSETS[st])):
                    upd = upd + recv_ref[q, st, p]
                acc_ref[q] = upd

            def finalize(q, scale, shift):
                mean = acc_ref[q, 0].astype(jnp.float32) / c_global
                var = (acc_ref[q, 1].astype(jnp.float32) / c_global
                       - mean * mean)
                inv = lax.rsqrt(var + EPS)
                xq = xv_ref[q]
                h = (xq - mean[:, :, None]) * inv[:, :, None]
                ov_ref[q] = h * (1.0 + scale[:, None, :]) + shift[:, None, :]
                cp = pltpu.make_async_copy(
                    ov_ref.at[q], chunk_slice(out_ref, q), store_sems.at[q])
                cp.start()
                return cp

            stats(0)
            pl.semaphore_wait(barrier_sem, len(NEAR_OFFSETS))
            inflight = {0: start_stage(0, 0)}
            for q in range(1, N_CHUNKS):
                stats(q)
                inflight[q] = start_stage(q, 0)
            scale = jnp.dot(t_ref[...], ws_ref[...],
                            preferred_element_type=jnp.float32)
            shift = jnp.dot(t_ref[...], wsh_ref[...],
                            preferred_element_type=jnp.float32)
            for st in range(1, N_STAGES):
                for q in range(N_CHUNKS):
                    finish_stage(q, st - 1, inflight[q])
                    if st == 1 and q == 0:
                        pl.semaphore_wait(far_sem, len(FAR_OFFSETS))
                    inflight[q] = start_stage(q, st)
            stores = []
            for q in range(N_CHUNKS):
                finish_stage(q, N_STAGES - 1, inflight[q])
                stores.append(finalize(q, scale, shift))
            for cp in stores:
                cp.wait()

    return pl.pallas_call(
        body,
        out_shape=jax.ShapeDtypeStruct((b, s, c), jnp.float32),
        in_specs=[
            pl.BlockSpec(memory_space=pltpu.HBM),
            pl.BlockSpec(memory_space=pltpu.VMEM),
            pl.BlockSpec(memory_space=pltpu.VMEM),
            pl.BlockSpec(memory_space=pltpu.VMEM),
        ],
        out_specs=pl.BlockSpec(memory_space=pltpu.HBM),
        scratch_shapes=[
            pltpu.VMEM((N_CHUNKS, b, s2, c), jnp.float32),
            pltpu.VMEM((N_CHUNKS, b, s2, c), jnp.float32),
            pltpu.VMEM((N_CHUNKS, 2, b, s2), jnp.bfloat16),
            pltpu.VMEM((N_CHUNKS, N_STAGES, MAX_RADIX, 2, b, s2),
                       jnp.bfloat16),
            pltpu.SemaphoreType.DMA((N_CHUNKS,)),
            pltpu.SemaphoreType.DMA((N_CHUNKS,)),
            pltpu.SemaphoreType.DMA((N_CHUNKS, N_STAGES, MAX_RADIX)),
            pltpu.SemaphoreType.DMA((N_CHUNKS, N_STAGES, MAX_RADIX)),
        ],
        compiler_params=pltpu.CompilerParams(collective_id=0),
    )(x, t_emb, W_scale, W_shift)


# device time: 22494 ns/iter; 1.0241x vs baseline; 1.0241x over previous
import jax
import jax.numpy as jnp
from jax import lax
from jax.experimental import pallas as pl
from jax.experimental.pallas import tpu as pltpu

N_DEV = 32
STAGE_OFFSETS = ((1, 2, 3), (4, 8, 12), (16,))
ALL_OFFSETS = tuple(o for offs in STAGE_OFFSETS for o in offs)
N_STAGES = len(STAGE_OFFSETS)
MAX_RADIX = 3
N_CHUNKS = 2
EPS = 1e-5


def kernel(x, t_emb, W_scale, W_shift):
    b, s, c = x.shape
    c_global = c * N_DEV
    s2 = s // N_CHUNKS

    def body(x_ref, t_ref, ws_ref, wsh_ref, out_ref,
             acc_ref, recv_ref, send_sems, recv_sems):
        my = lax.axis_index("i")

        barrier_sem = pltpu.get_barrier_semaphore()
        for off in ALL_OFFSETS:
            pl.semaphore_signal(
                barrier_sem, inc=1,
                device_id=(my ^ off,),
                device_id_type=pl.DeviceIdType.MESH,
            )

        def stats(q):
            xq = x_ref[:, q * s2:(q + 1) * s2, :]
            acc_ref[q, 0] = jnp.sum(xq, axis=-1).astype(jnp.bfloat16)
            acc_ref[q, 1] = jnp.sum(xq * xq, axis=-1).astype(jnp.bfloat16)

        def start_stage(q, st):
            rdmas = []
            for p, off in enumerate(STAGE_OFFSETS[st]):
                rdma = pltpu.make_async_remote_copy(
                    src_ref=acc_ref.at[q],
                    dst_ref=recv_ref.at[q, st, p],
                    send_sem=send_sems.at[q, st, p],
                    recv_sem=recv_sems.at[q, st, p],
                    device_id=(my ^ off,),
                    device_id_type=pl.DeviceIdType.MESH,
                )
                rdma.start()
                rdmas.append(rdma)
            return rdmas

        def finish_stage(q, st, rdmas):
            for rdma in rdmas:
                rdma.wait()
            upd = acc_ref[q]
            for p in range(len(STAGE_OFFSETS[st])):
                upd = upd + recv_ref[q, st, p]
            acc_ref[q] = upd

        def finalize(q, scale, shift):
            mean = acc_ref[q, 0].astype(jnp.float32) / c_global
            var = acc_ref[q, 1].astype(jnp.float32) / c_global - mean * mean
            inv = lax.rsqrt(var + EPS)
            xq = x_ref[:, q * s2:(q + 1) * s2, :]
            h = (xq - mean[:, :, None]) * inv[:, :, None]
            out_ref[:, q * s2:(q + 1) * s2, :] = (
                h * (1.0 + scale[:, None, :]) + shift[:, None, :]
            )

        stats(0)
        pl.semaphore_wait(barrier_sem, len(ALL_OFFSETS))
        a0 = start_stage(0, 0)
        stats(1)
        b0 = start_stage(1, 0)
        scale = jnp.dot(t_ref[...], ws_ref[...],
                        preferred_element_type=jnp.float32)
        shift = jnp.dot(t_ref[...], wsh_ref[...],
                        preferred_element_type=jnp.float32)
        finish_stage(0, 0, a0)
        a1 = start_stage(0, 1)
        finish_stage(1, 0, b0)
        b1 = start_stage(1, 1)
        finish_stage(0, 1, a1)
        a2 = start_stage(0, 2)
        finish_stage(1, 1, b1)
        b2 = start_stage(1, 2)
        finish_stage(0, 2, a2)
        finalize(0, scale, shift)
        finish_stage(1, 2, b2)
        finalize(1, scale, shift)

    return pl.pallas_call(
        body,
        out_shape=jax.ShapeDtypeStruct((b, s, c), jnp.float32),
        in_specs=[pl.BlockSpec(memory_space=pltpu.VMEM)] * 4,
        out_specs=pl.BlockSpec(memory_space=pltpu.VMEM),
        scratch_shapes=[
            pltpu.VMEM((N_CHUNKS, 2, b, s2), jnp.bfloat16),
            pltpu.VMEM((N_CHUNKS, N_STAGES, MAX_RADIX, 2, b, s2),
                       jnp.bfloat16),
            pltpu.SemaphoreType.DMA((N_CHUNKS, N_STAGES, MAX_RADIX)),
            pltpu.SemaphoreType.DMA((N_CHUNKS, N_STAGES, MAX_RADIX)),
        ],
        compiler_params=pltpu.CompilerParams(collective_id=0),
    )(x, t_emb, W_scale, W_shift)
